# baseline (speedup 1.0000x reference)
import functools

import jax
import jax.numpy as jnp
from jax import lax
from jax.experimental import pallas as pl
from jax.experimental.pallas import tpu as pltpu
from jax.experimental.pallas import tpu_sc as plsc

N = 10000
NP = 10240
E = 320000
D = 128
NC = 2
NS = 16
NW = NC * NS
CH = 80
NCHUNK = 125
EPW = NCHUNK * CH
RPW = NP // NS

_mesh = plsc.VectorSubcoreMesh(core_axis_name="c", subcore_axis_name="s")


@functools.partial(
    pl.kernel,
    mesh=_mesh,
    out_type=jax.ShapeDtypeStruct((NC, NS, RPW), jnp.float32),
    scratch_types=[
        pltpu.VMEM((NCHUNK, CH), jnp.int32),
        pltpu.VMEM((CH,), jnp.float32),
        pltpu.VMEM_SHARED((NP,), jnp.float32),
    ],
)
def _deg_kernel(dst_hbm, ones_hbm, zeros_hbm, out_hbm, idx_v, ones_v, acc_sh):
    cid = lax.axis_index("c")
    sid = lax.axis_index("s")
    wid = sid * NC + cid
    pltpu.sync_copy(dst_hbm.at[wid], idx_v)
    pltpu.sync_copy(ones_hbm, ones_v)
    pltpu.sync_copy(zeros_hbm, acc_sh.at[pl.ds(sid * RPW, RPW)])
    plsc.subcore_barrier()

    def body(j, carry):
        pltpu.sync_copy(ones_v, acc_sh.at[idx_v.at[j]], add=True)
        return carry

    lax.fori_loop(0, NCHUNK, body, 0)
    plsc.subcore_barrier()
    pltpu.sync_copy(acc_sh.at[pl.ds(sid * RPW, RPW)], out_hbm.at[cid, sid])


@functools.partial(
    pl.kernel,
    mesh=_mesh,
    out_type=jax.ShapeDtypeStruct((NC, NS, RPW, D), jnp.float32),
    scratch_types=[
        pltpu.VMEM((CH,), jnp.int32),
        pltpu.VMEM((CH,), jnp.int32),
        pltpu.VMEM((CH,), jnp.int32),
        pltpu.VMEM((CH,), jnp.int32),
        pltpu.VMEM((CH,), jnp.int32),
        pltpu.VMEM((CH,), jnp.int32),
        pltpu.VMEM((CH,), jnp.int32),
        pltpu.VMEM((CH,), jnp.int32),
        pltpu.VMEM((CH, D), jnp.float32),
        pltpu.VMEM((CH, D), jnp.float32),
        pltpu.VMEM((CH, D), jnp.float32),
        pltpu.VMEM_SHARED((NP, D), jnp.float32),
        pltpu.SemaphoreType.DMA,
        pltpu.SemaphoreType.DMA,
        pltpu.SemaphoreType.DMA,
        pltpu.SemaphoreType.DMA,
        pltpu.SemaphoreType.DMA,
        pltpu.SemaphoreType.DMA,
        pltpu.SemaphoreType.DMA,
        pltpu.SemaphoreType.DMA,
        pltpu.SemaphoreType.DMA,
        pltpu.SemaphoreType.DMA,
        pltpu.SemaphoreType.DMA,
        pltpu.SemaphoreType.DMA,
        pltpu.SemaphoreType.DMA,
        pltpu.SemaphoreType.DMA,
    ],
)
def _spmm_kernel(src_hbm, dst_hbm, y_hbm, zeros_hbm, out_hbm,
                 s0, s1, s2, s3, d0, d1, d2, d3, ra, rb, rc, acc_sh,
                 ps0, ps1, ps2, ps3, pd0, pd1, pd2, pd3,
                 ga, gb, gc, wa, wb, wc):
    cid = lax.axis_index("c")
    sid = lax.axis_index("s")
    wid = sid * NC + cid
    base = wid * EPW
    sv = [s0, s1, s2, s3]
    dv = [d0, d1, d2, d3]
    psem = [ps0, ps1, ps2, ps3]
    pdem = [pd0, pd1, pd2, pd3]
    rows = [ra, rb, rc]
    gsem = [ga, gb, gc]
    ssem = [wa, wb, wc]

    pltpu.sync_copy(zeros_hbm, acc_sh.at[pl.ds(sid * RPW, RPW)])
    plsc.subcore_barrier()

    def fire_idx(j, k4):
        pltpu.async_copy(src_hbm.at[pl.ds(base + j * CH, CH)], sv[k4], psem[k4])
        pltpu.async_copy(dst_hbm.at[pl.ds(base + j * CH, CH)], dv[k4], pdem[k4])

    def wait_idx(buf, sem):
        pltpu.make_async_copy(src_hbm.at[pl.ds(0, CH)], buf, sem).wait()

    def wait_rows(buf, sem):
        pltpu.make_async_copy(y_hbm.at[s0], buf, sem).wait()

    def wait_scat(buf, sem):
        pltpu.make_async_copy(buf, acc_sh.at[d0], sem).wait()

    def step(t, m3, m4, s_wait, do_g, do_i):
        n3, n4 = (m3 + 1) % 3, (m4 + 1) % 4
        wait_idx(dv[m4], pdem[m4])
        wait_rows(rows[m3], gsem[m3])
        pltpu.async_copy(rows[m3], acc_sh.at[dv[m4]], ssem[m3], add=True)
        if s_wait:
            wait_scat(rows[n3], ssem[n3])
        if do_g:
            wait_idx(sv[n4], psem[n4])
            pltpu.async_copy(y_hbm.at[sv[n4]], rows[n3], gsem[n3])
        if do_i:
            fire_idx(t + 2, (m4 + 2) % 4)

    fire_idx(0, 0)
    fire_idx(1, 1)
    wait_idx(s0, ps0)
    pltpu.async_copy(y_hbm.at[s0], ra, ga)

    step(0, 0, 0, False, True, True)
    step(1, 1, 1, False, True, True)

    def body(u, carry):
        t = 12 * u + 2
        for k in range(12):
            step(t + k, (2 + k) % 3, (2 + k) % 4, True, True, True)
        return carry

    lax.fori_loop(0, 10, body, 0)
    step(122, 122 % 3, 122 % 4, True, True, True)
    step(123, 123 % 3, 123 % 4, True, True, False)
    step(124, 124 % 3, 124 % 4, True, False, False)
    wait_scat(rows[123 % 3], ssem[123 % 3])
    wait_scat(rows[124 % 3], ssem[124 % 3])
    plsc.subcore_barrier()
    pltpu.sync_copy(acc_sh.at[pl.ds(sid * RPW, RPW)], out_hbm.at[cid, sid])


_ROWS = 2048
_GRID = NP // _ROWS


def _dinv_block(deg_ref):
    d = deg_ref[0] + deg_ref[1]
    return jnp.broadcast_to(lax.rsqrt(1.0 + d), (_ROWS, D))


def _mm1_body(h_ref, w_ref, deg_ref, y1_ref):
    xw = jnp.dot(h_ref[...], w_ref[...], preferred_element_type=jnp.float32)
    y1_ref[...] = xw * _dinv_block(deg_ref)


def _mid_body(acc_ref, y_ref, deg_ref, b_ref, w_ref, y2_ref):
    dinv = _dinv_block(deg_ref)
    s = acc_ref[0] + acc_ref[1] + y_ref[...]
    x2 = jnp.sin(s * dinv + b_ref[...])
    xw = jnp.dot(x2, w_ref[...], preferred_element_type=jnp.float32)
    y2_ref[...] = xw * dinv


def _final_body(acc_ref, y_ref, deg_ref, b_ref, out_ref):
    s = acc_ref[0] + acc_ref[1] + y_ref[...]
    out_ref[...] = jnp.sin(s * _dinv_block(deg_ref) + b_ref[...])


def _row_spec():
    return pl.BlockSpec((_ROWS, D), lambda i: (i, 0))


def _acc_spec():
    return pl.BlockSpec((NC, _ROWS, D), lambda i: (0, i, 0))


def _deg_spec():
    return pl.BlockSpec((NC, _ROWS, 1), lambda i: (0, i, 0))


def _full_spec(shape):
    nd = len(shape)
    return pl.BlockSpec(shape, lambda i: (0,) * nd)


def _mm1(h2, W1, deg3):
    return pl.pallas_call(
        _mm1_body,
        grid=(_GRID,),
        in_specs=[_row_spec(), _full_spec((D, D)), _deg_spec()],
        out_specs=_row_spec(),
        out_shape=jax.ShapeDtypeStruct((NP, D), jnp.float32),
    )(h2, W1, deg3)


def _mid(acc, y1, deg3, b1, W2):
    return pl.pallas_call(
        _mid_body,
        grid=(_GRID,),
        in_specs=[_acc_spec(), _row_spec(), _deg_spec(),
                  _full_spec((1, D)), _full_spec((D, D))],
        out_specs=_row_spec(),
        out_shape=jax.ShapeDtypeStruct((NP, D), jnp.float32),
    )(acc, y1, deg3, b1, W2)


def _final(acc, y2, deg3, b2):
    return pl.pallas_call(
        _final_body,
        grid=(_GRID,),
        in_specs=[_acc_spec(), _row_spec(), _deg_spec(), _full_spec((1, D))],
        out_specs=_row_spec(),
        out_shape=jax.ShapeDtypeStruct((NP, D), jnp.float32),
    )(acc, y2, deg3, b2)


def kernel(t, h, edge_index, W1, b1, W2, b2):
    src = edge_index[0]
    dst = edge_index[1]
    dst3 = dst.reshape(NW, NCHUNK, CH)
    zeros_d = jnp.zeros((RPW, D), jnp.float32)
    zeros1 = jnp.zeros((RPW,), jnp.float32)
    ones1 = jnp.ones((CH,), jnp.float32)
    h2 = jnp.zeros((NP, D), jnp.float32).at[:N].set(h.reshape(N, D))
    b1r = b1.reshape(1, D)
    b2r = b2.reshape(1, D)

    deg = _deg_kernel(dst3, ones1, zeros1)
    deg3 = deg.reshape(NC, NP, 1)

    y1 = _mm1(h2, W1, deg3)
    acc1 = _spmm_kernel(src, dst, y1, zeros_d).reshape(NC, NP, D)
    y2 = _mid(acc1, y1, deg3, b1r, W2)
    acc2 = _spmm_kernel(src, dst, y2, zeros_d).reshape(NC, NP, D)
    out = _final(acc2, y2, deg3, b2r)
    return out[:N].reshape(1, N, D)


# speedup vs baseline: 1.0042x; 1.0042x over previous
import functools

import jax
import jax.numpy as jnp
from jax import lax
from jax.experimental import pallas as pl
from jax.experimental.pallas import tpu as pltpu
from jax.experimental.pallas import tpu_sc as plsc

N = 10000
NP = 10240
E = 320000
D = 128
NC = 2
NS = 16
NW = NC * NS
CH = 80
NCHUNK = 125
EPW = NCHUNK * CH
RPW = NP // NS

_mesh = plsc.VectorSubcoreMesh(core_axis_name="c", subcore_axis_name="s")


@functools.partial(
    pl.kernel,
    mesh=_mesh,
    out_type=jax.ShapeDtypeStruct((NC, NS, RPW), jnp.float32),
    scratch_types=[
        pltpu.VMEM((NCHUNK, CH), jnp.int32),
        pltpu.VMEM((CH,), jnp.float32),
        pltpu.VMEM_SHARED((NP,), jnp.float32),
    ],
)
def _deg_kernel(dst_hbm, ones_hbm, zeros_hbm, out_hbm, idx_v, ones_v, acc_sh):
    cid = lax.axis_index("c")
    sid = lax.axis_index("s")
    wid = sid * NC + cid
    pltpu.sync_copy(dst_hbm.at[wid], idx_v)
    pltpu.sync_copy(ones_hbm, ones_v)
    pltpu.sync_copy(zeros_hbm, acc_sh.at[pl.ds(sid * RPW, RPW)])
    plsc.subcore_barrier()

    def body(j, carry):
        pltpu.sync_copy(ones_v, acc_sh.at[idx_v.at[j]], add=True)
        return carry

    lax.fori_loop(0, NCHUNK, body, 0)
    plsc.subcore_barrier()
    pltpu.sync_copy(acc_sh.at[pl.ds(sid * RPW, RPW)], out_hbm.at[cid, sid])


@functools.partial(
    pl.kernel,
    mesh=_mesh,
    out_type=jax.ShapeDtypeStruct((NC, NS, RPW, D), jnp.float32),
    scratch_types=[
        pltpu.VMEM((EPW,), jnp.int32),
        pltpu.VMEM((NCHUNK, CH), jnp.int32),
        pltpu.VMEM((CH, D), jnp.float32),
        pltpu.VMEM((CH, D), jnp.float32),
        pltpu.VMEM_SHARED((NP, D), jnp.float32),
        pltpu.SemaphoreType.DMA,
        pltpu.SemaphoreType.DMA,
        pltpu.SemaphoreType.DMA,
        pltpu.SemaphoreType.DMA,
    ],
)
def _spmm_kernel(src_hbm, dst_hbm, y_hbm, zeros_hbm, out_hbm,
                 src_v, dst_v, ra, rb, acc_sh, ga, gb, wa, wb):
    cid = lax.axis_index("c")
    sid = lax.axis_index("s")
    wid = sid * NC + cid
    rows = [ra, rb]
    gsem = [ga, gb]
    ssem = [wa, wb]

    pltpu.sync_copy(src_hbm.at[wid], src_v)
    pltpu.sync_copy(dst_hbm.at[wid], dst_v)
    pltpu.sync_copy(zeros_hbm, acc_sh.at[pl.ds(sid * RPW, RPW)])
    plsc.subcore_barrier()

    def wait_rows(buf, sem):
        pltpu.make_async_copy(y_hbm.at[src_v.at[pl.ds(0, CH)]], buf, sem).wait()

    def wait_scat(buf, sem):
        pltpu.make_async_copy(buf, acc_sh.at[dst_v.at[0]], sem).wait()

    def step(t, m2, s_wait, do_g):
        n2 = 1 - m2
        wait_rows(rows[m2], gsem[m2])
        pltpu.async_copy(rows[m2], acc_sh.at[dst_v.at[t]], ssem[m2], add=True)
        if s_wait:
            wait_scat(rows[n2], ssem[n2])
        if do_g:
            pltpu.async_copy(y_hbm.at[src_v.at[pl.ds((t + 1) * CH, CH)]],
                             rows[n2], gsem[n2])

    pltpu.async_copy(y_hbm.at[src_v.at[pl.ds(0, CH)]], ra, ga)
    step(0, 0, False, True)

    def body(u, carry):
        t = 2 * u + 1
        step(t, 1, True, True)
        step(t + 1, 0, True, True)
        return carry

    lax.fori_loop(0, 61, body, 0)
    step(123, 1, True, True)
    step(124, 0, True, False)
    wait_scat(rows[0], ssem[0])
    plsc.subcore_barrier()
    pltpu.sync_copy(acc_sh.at[pl.ds(sid * RPW, RPW)], out_hbm.at[cid, sid])


_ROWS = 2048
_GRID = NP // _ROWS


def _dinv_block(deg_ref):
    d = deg_ref[0] + deg_ref[1]
    return jnp.broadcast_to(lax.rsqrt(1.0 + d), (_ROWS, D))


def _mm1_body(h_ref, w_ref, deg_ref, y1_ref):
    xw = jnp.dot(h_ref[...], w_ref[...], preferred_element_type=jnp.float32)
    y1_ref[...] = xw * _dinv_block(deg_ref)


def _mid_body(acc_ref, y_ref, deg_ref, b_ref, w_ref, y2_ref):
    dinv = _dinv_block(deg_ref)
    s = acc_ref[0] + acc_ref[1] + y_ref[...]
    x2 = jnp.sin(s * dinv + b_ref[...])
    xw = jnp.dot(x2, w_ref[...], preferred_element_type=jnp.float32)
    y2_ref[...] = xw * dinv


def _final_body(acc_ref, y_ref, deg_ref, b_ref, out_ref):
    s = acc_ref[0] + acc_ref[1] + y_ref[...]
    out_ref[...] = jnp.sin(s * _dinv_block(deg_ref) + b_ref[...])


def _row_spec():
    return pl.BlockSpec((_ROWS, D), lambda i: (i, 0))


def _acc_spec():
    return pl.BlockSpec((NC, _ROWS, D), lambda i: (0, i, 0))


def _deg_spec():
    return pl.BlockSpec((NC, _ROWS, 1), lambda i: (0, i, 0))


def _full_spec(shape):
    nd = len(shape)
    return pl.BlockSpec(shape, lambda i: (0,) * nd)


def _mm1(h2, W1, deg3):
    return pl.pallas_call(
        _mm1_body,
        grid=(_GRID,),
        in_specs=[_row_spec(), _full_spec((D, D)), _deg_spec()],
        out_specs=_row_spec(),
        out_shape=jax.ShapeDtypeStruct((NP, D), jnp.float32),
    )(h2, W1, deg3)


def _mid(acc, y1, deg3, b1, W2):
    return pl.pallas_call(
        _mid_body,
        grid=(_GRID,),
        in_specs=[_acc_spec(), _row_spec(), _deg_spec(),
                  _full_spec((1, D)), _full_spec((D, D))],
        out_specs=_row_spec(),
        out_shape=jax.ShapeDtypeStruct((NP, D), jnp.float32),
    )(acc, y1, deg3, b1, W2)


def _final(acc, y2, deg3, b2):
    return pl.pallas_call(
        _final_body,
        grid=(_GRID,),
        in_specs=[_acc_spec(), _row_spec(), _deg_spec(), _full_spec((1, D))],
        out_specs=_row_spec(),
        out_shape=jax.ShapeDtypeStruct((NP, D), jnp.float32),
    )(acc, y2, deg3, b2)


def kernel(t, h, edge_index, W1, b1, W2, b2):
    src2 = edge_index[0].reshape(NW, EPW)
    dst3 = edge_index[1].reshape(NW, NCHUNK, CH)
    zeros_d = jnp.zeros((RPW, D), jnp.float32)
    zeros1 = jnp.zeros((RPW,), jnp.float32)
    ones1 = jnp.ones((CH,), jnp.float32)
    h2 = jnp.zeros((NP, D), jnp.float32).at[:N].set(h.reshape(N, D))
    b1r = b1.reshape(1, D)
    b2r = b2.reshape(1, D)

    deg = _deg_kernel(dst3, ones1, zeros1)
    deg3 = deg.reshape(NC, NP, 1)

    y1 = _mm1(h2, W1, deg3)
    acc1 = _spmm_kernel(src2, dst3, y1, zeros_d).reshape(NC, NP, D)
    y2 = _mid(acc1, y1, deg3, b1r, W2)
    acc2 = _spmm_kernel(src2, dst3, y2, zeros_d).reshape(NC, NP, D)
    out = _final(acc2, y2, deg3, b2r)
    return out[:N].reshape(1, N, D)


# speedup vs baseline: 1.0899x; 1.0854x over previous
import functools

import jax
import jax.numpy as jnp
from jax import lax
from jax.experimental import pallas as pl
from jax.experimental.pallas import tpu as pltpu
from jax.experimental.pallas import tpu_sc as plsc

N = 10000
NP = 10240
E = 320000
D = 128
NC = 2
NS = 16
NW = NC * NS
CH = 104
NCHUNK = 98
EPW = NCHUNK * CH
EPAD = NW * EPW
RPW = NP // NS

_mesh = plsc.VectorSubcoreMesh(core_axis_name="c", subcore_axis_name="s")


@functools.partial(
    pl.kernel,
    mesh=_mesh,
    out_type=jax.ShapeDtypeStruct((NC, NS, RPW), jnp.float32),
    scratch_types=[
        pltpu.VMEM((NCHUNK, CH), jnp.int32),
        pltpu.VMEM((CH,), jnp.float32),
        pltpu.VMEM_SHARED((NP,), jnp.float32),
    ],
)
def _deg_kernel(dst_hbm, ones_hbm, zeros_hbm, out_hbm, idx_v, ones_v, acc_sh):
    cid = lax.axis_index("c")
    sid = lax.axis_index("s")
    wid = sid * NC + cid
    pltpu.sync_copy(dst_hbm.at[wid], idx_v)
    pltpu.sync_copy(ones_hbm, ones_v)
    pltpu.sync_copy(zeros_hbm, acc_sh.at[pl.ds(sid * RPW, RPW)])
    plsc.subcore_barrier()

    def body(j, carry):
        pltpu.sync_copy(ones_v, acc_sh.at[idx_v.at[j]], add=True)
        return carry

    lax.fori_loop(0, NCHUNK, body, 0)
    plsc.subcore_barrier()
    pltpu.sync_copy(acc_sh.at[pl.ds(sid * RPW, RPW)], out_hbm.at[cid, sid])


@functools.partial(
    pl.kernel,
    mesh=_mesh,
    out_type=jax.ShapeDtypeStruct((NC, NS, RPW, D), jnp.float32),
    scratch_types=[
        pltpu.VMEM((CH,), jnp.int32),
        pltpu.VMEM((CH,), jnp.int32),
        pltpu.VMEM((CH,), jnp.int32),
        pltpu.VMEM((CH,), jnp.int32),
        pltpu.VMEM((CH,), jnp.int32),
        pltpu.VMEM((CH,), jnp.int32),
        pltpu.VMEM((CH, D), jnp.float32),
        pltpu.VMEM((CH, D), jnp.float32),
        pltpu.VMEM_SHARED((NP, D), jnp.float32),
        pltpu.SemaphoreType.DMA,
        pltpu.SemaphoreType.DMA,
        pltpu.SemaphoreType.DMA,
        pltpu.SemaphoreType.DMA,
        pltpu.SemaphoreType.DMA,
        pltpu.SemaphoreType.DMA,
        pltpu.SemaphoreType.DMA,
        pltpu.SemaphoreType.DMA,
        pltpu.SemaphoreType.DMA,
        pltpu.SemaphoreType.DMA,
    ],
)
def _spmm_kernel(src_hbm, dst_hbm, y_hbm, zeros_hbm, out_hbm,
                 s0, s1, s2, d0, d1, d2, ra, rb, acc_sh,
                 ps0, ps1, ps2, pd0, pd1, pd2, ga, gb, wa, wb):
    cid = lax.axis_index("c")
    sid = lax.axis_index("s")
    wid = sid * NC + cid
    sv = [s0, s1, s2]
    dv = [d0, d1, d2]
    psem = [ps0, ps1, ps2]
    pdem = [pd0, pd1, pd2]
    rows = [ra, rb]
    gsem = [ga, gb]
    ssem = [wa, wb]

    pltpu.sync_copy(zeros_hbm, acc_sh.at[pl.ds(sid * RPW, RPW)])
    plsc.subcore_barrier()

    def fire_idx(j, k3):
        pltpu.async_copy(src_hbm.at[wid, j], sv[k3], psem[k3])
        pltpu.async_copy(dst_hbm.at[wid, j], dv[k3], pdem[k3])

    def wait_idx(buf, sem):
        pltpu.make_async_copy(src_hbm.at[wid, 0], buf, sem).wait()

    def wait_rows(buf, sem):
        pltpu.make_async_copy(y_hbm.at[s0], buf, sem).wait()

    def wait_scat(buf, sem):
        pltpu.make_async_copy(buf, acc_sh.at[d0], sem).wait()

    def step(t, m2, m3, s_wait, do_g, do_i):
        n2, n3 = (m2 + 1) % 2, (m3 + 1) % 3
        wait_idx(dv[m3], pdem[m3])
        wait_rows(rows[m2], gsem[m2])
        pltpu.async_copy(rows[m2], acc_sh.at[dv[m3]], ssem[m2], add=True)
        if s_wait:
            wait_scat(rows[n2], ssem[n2])
        if do_g:
            wait_idx(sv[n3], psem[n3])
            pltpu.async_copy(y_hbm.at[sv[n3]], rows[n2], gsem[n2])
        if do_i:
            fire_idx(t + 2, (m3 + 2) % 3)

    fire_idx(0, 0)
    fire_idx(1, 1)
    wait_idx(s0, ps0)
    pltpu.async_copy(y_hbm.at[s0], ra, ga)

    step(0, 0, 0, False, True, True)

    def body(u, carry):
        t = 6 * u + 1
        step(t, 1, 1, True, True, True)
        step(t + 1, 0, 2, True, True, True)
        step(t + 2, 1, 0, True, True, True)
        step(t + 3, 0, 1, True, True, True)
        step(t + 4, 1, 2, True, True, True)
        step(t + 5, 0, 0, True, True, True)
        return carry

    lax.fori_loop(0, 15, body, 0)
    for t in range(91, 96):
        step(t, t % 2, t % 3, True, True, True)
    step(96, 0, 0, True, True, False)
    step(97, 1, 1, True, False, False)
    wait_scat(rows[1], ssem[1])
    plsc.subcore_barrier()
    pltpu.sync_copy(acc_sh.at[pl.ds(sid * RPW, RPW)], out_hbm.at[cid, sid])


_ROWS = 2048
_GRID = NP // _ROWS


def _dinv_block(deg_ref):
    d = deg_ref[0] + deg_ref[1]
    return jnp.broadcast_to(lax.rsqrt(1.0 + d), (_ROWS, D))


def _mm1_body(h_ref, w_ref, deg_ref, y1_ref):
    xw = jnp.dot(h_ref[...], w_ref[...], preferred_element_type=jnp.float32)
    y1_ref[...] = xw * _dinv_block(deg_ref)


def _mid_body(acc_ref, y_ref, deg_ref, b_ref, w_ref, y2_ref):
    dinv = _dinv_block(deg_ref)
    s = acc_ref[0] + acc_ref[1] + y_ref[...]
    x2 = jnp.sin(s * dinv + b_ref[...])
    xw = jnp.dot(x2, w_ref[...], preferred_element_type=jnp.float32)
    y2_ref[...] = xw * dinv


def _final_body(acc_ref, y_ref, deg_ref, b_ref, out_ref):
    s = acc_ref[0] + acc_ref[1] + y_ref[...]
    out_ref[...] = jnp.sin(s * _dinv_block(deg_ref) + b_ref[...])


def _row_spec():
    return pl.BlockSpec((_ROWS, D), lambda i: (i, 0))


def _acc_spec():
    return pl.BlockSpec((NC, _ROWS, D), lambda i: (0, i, 0))


def _deg_spec():
    return pl.BlockSpec((NC, _ROWS, 1), lambda i: (0, i, 0))


def _full_spec(shape):
    nd = len(shape)
    return pl.BlockSpec(shape, lambda i: (0,) * nd)


def _mm1(h2, W1, deg3):
    return pl.pallas_call(
        _mm1_body,
        grid=(_GRID,),
        in_specs=[_row_spec(), _full_spec((D, D)), _deg_spec()],
        out_specs=_row_spec(),
        out_shape=jax.ShapeDtypeStruct((NP, D), jnp.float32),
    )(h2, W1, deg3)


def _mid(acc, y1, deg3, b1, W2):
    return pl.pallas_call(
        _mid_body,
        grid=(_GRID,),
        in_specs=[_acc_spec(), _row_spec(), _deg_spec(),
                  _full_spec((1, D)), _full_spec((D, D))],
        out_specs=_row_spec(),
        out_shape=jax.ShapeDtypeStruct((NP, D), jnp.float32),
    )(acc, y1, deg3, b1, W2)


def _final(acc, y2, deg3, b2):
    return pl.pallas_call(
        _final_body,
        grid=(_GRID,),
        in_specs=[_acc_spec(), _row_spec(), _deg_spec(), _full_spec((1, D))],
        out_specs=_row_spec(),
        out_shape=jax.ShapeDtypeStruct((NP, D), jnp.float32),
    )(acc, y2, deg3, b2)


def kernel(t, h, edge_index, W1, b1, W2, b2):
    pad = jnp.arange(EPAD - E, dtype=jnp.int32)
    src3 = jnp.concatenate([edge_index[0], pad % N]).reshape(NW, NCHUNK, CH)
    dst3 = jnp.concatenate([edge_index[1], N + pad % (NP - N)]
                           ).reshape(NW, NCHUNK, CH)
    zeros_d = jnp.zeros((RPW, D), jnp.float32)
    zeros1 = jnp.zeros((RPW,), jnp.float32)
    ones1 = jnp.ones((CH,), jnp.float32)
    h2 = h.reshape(N, D)
    b1r = b1.reshape(1, D)
    b2r = b2.reshape(1, D)

    deg = _deg_kernel(dst3, ones1, zeros1)
    deg3 = deg.reshape(NC, NP, 1)

    y1 = _mm1(h2, W1, deg3)
    acc1 = _spmm_kernel(src3, dst3, y1, zeros_d).reshape(NC, NP, D)
    y2 = _mid(acc1, y1, deg3, b1r, W2)
    acc2 = _spmm_kernel(src3, dst3, y2, zeros_d).reshape(NC, NP, D)
    out = _final(acc2, y2, deg3, b2r)
    return out[:N].reshape(1, N, D)


# speedup vs baseline: 1.0956x; 1.0052x over previous
import functools

import jax
import jax.numpy as jnp
from jax import lax
from jax.experimental import pallas as pl
from jax.experimental.pallas import tpu as pltpu
from jax.experimental.pallas import tpu_sc as plsc

N = 10000
NP = 10240
E = 320000
D = 128
NC = 2
NS = 16
NW = NC * NS
CH = 104
NF = 96
CT = 16
EPW = E // NW
CHD = 80
NCHD = 125
RPW = NP // NS

_mesh = plsc.VectorSubcoreMesh(core_axis_name="c", subcore_axis_name="s")


@functools.partial(
    pl.kernel,
    mesh=_mesh,
    out_type=jax.ShapeDtypeStruct((NC, NS, RPW), jnp.float32),
    scratch_types=[
        pltpu.VMEM((NCHD, CHD), jnp.int32),
        pltpu.VMEM((CHD,), jnp.float32),
        pltpu.VMEM_SHARED((NP,), jnp.float32),
    ],
)
def _deg_kernel(dst_hbm, ones_hbm, zeros_hbm, out_hbm, idx_v, ones_v, acc_sh):
    cid = lax.axis_index("c")
    sid = lax.axis_index("s")
    wid = sid * NC + cid
    pltpu.sync_copy(dst_hbm.at[wid], idx_v)
    pltpu.sync_copy(ones_hbm, ones_v)
    pltpu.sync_copy(zeros_hbm, acc_sh.at[pl.ds(sid * RPW, RPW)])
    plsc.subcore_barrier()

    def body(j, carry):
        pltpu.sync_copy(ones_v, acc_sh.at[idx_v.at[j]], add=True)
        return carry

    lax.fori_loop(0, NCHD, body, 0)
    plsc.subcore_barrier()
    pltpu.sync_copy(acc_sh.at[pl.ds(sid * RPW, RPW)], out_hbm.at[cid, sid])


@functools.partial(
    pl.kernel,
    mesh=_mesh,
    out_type=jax.ShapeDtypeStruct((NC, NS, RPW, D), jnp.float32),
    scratch_types=[
        pltpu.VMEM((CH,), jnp.int32),
        pltpu.VMEM((CH,), jnp.int32),
        pltpu.VMEM((CH,), jnp.int32),
        pltpu.VMEM((CH,), jnp.int32),
        pltpu.VMEM((CH,), jnp.int32),
        pltpu.VMEM((CH,), jnp.int32),
        pltpu.VMEM((CT,), jnp.int32),
        pltpu.VMEM((CT,), jnp.int32),
        pltpu.VMEM((CH, D), jnp.float32),
        pltpu.VMEM((CH, D), jnp.float32),
        pltpu.VMEM_SHARED((NP, D), jnp.float32),
        pltpu.SemaphoreType.DMA,
        pltpu.SemaphoreType.DMA,
        pltpu.SemaphoreType.DMA,
        pltpu.SemaphoreType.DMA,
        pltpu.SemaphoreType.DMA,
        pltpu.SemaphoreType.DMA,
        pltpu.SemaphoreType.DMA,
        pltpu.SemaphoreType.DMA,
        pltpu.SemaphoreType.DMA,
        pltpu.SemaphoreType.DMA,
        pltpu.SemaphoreType.DMA,
        pltpu.SemaphoreType.DMA,
    ],
)
def _spmm_kernel(src_hbm, dst_hbm, y_hbm, zeros_hbm, out_hbm,
                 s0, s1, s2, d0, d1, d2, st, dt, ra, rb, acc_sh,
                 ps0, ps1, ps2, pd0, pd1, pd2, ga, gb, wa, wb, pst, pdt):
    cid = lax.axis_index("c")
    sid = lax.axis_index("s")
    wid = sid * NC + cid
    base = wid * EPW
    sv = [s0, s1, s2]
    dv = [d0, d1, d2]
    psem = [ps0, ps1, ps2]
    pdem = [pd0, pd1, pd2]
    rows = [ra, rb]
    gsem = [ga, gb]
    ssem = [wa, wb]

    pltpu.sync_copy(zeros_hbm, acc_sh.at[pl.ds(sid * RPW, RPW)])
    plsc.subcore_barrier()

    def fire_idx(j, k3):
        pltpu.async_copy(src_hbm.at[pl.ds(base + j * CH, CH)], sv[k3], psem[k3])
        pltpu.async_copy(dst_hbm.at[pl.ds(base + j * CH, CH)], dv[k3], pdem[k3])

    def wait_idx(buf, sem):
        pltpu.make_async_copy(src_hbm.at[pl.ds(0, CH)], buf, sem).wait()

    def wait_rows(buf, sem):
        pltpu.make_async_copy(y_hbm.at[s0], buf, sem).wait()

    def wait_scat(buf, sem):
        pltpu.make_async_copy(buf, acc_sh.at[d0], sem).wait()

    def step(t, m2, m3, s_wait, do_g, do_i):
        n2, n3 = (m2 + 1) % 2, (m3 + 1) % 3
        wait_idx(dv[m3], pdem[m3])
        wait_rows(rows[m2], gsem[m2])
        pltpu.async_copy(rows[m2], acc_sh.at[dv[m3]], ssem[m2], add=True)
        if s_wait:
            wait_scat(rows[n2], ssem[n2])
        if do_g:
            wait_idx(sv[n3], psem[n3])
            pltpu.async_copy(y_hbm.at[sv[n3]], rows[n2], gsem[n2])
        if do_i:
            fire_idx(t + 2, (m3 + 2) % 3)

    fire_idx(0, 0)
    fire_idx(1, 1)
    wait_idx(s0, ps0)
    pltpu.async_copy(y_hbm.at[s0], ra, ga)

    step(0, 0, 0, False, True, True)

    def body(u, carry):
        t = 6 * u + 1
        step(t, 1, 1, True, True, True)
        step(t + 1, 0, 2, True, True, True)
        step(t + 2, 1, 0, True, True, True)
        step(t + 3, 0, 1, True, True, True)
        step(t + 4, 1, 2, True, True, True)
        step(t + 5, 0, 0, True, True, True)
        return carry

    lax.fori_loop(0, 15, body, 0)
    for t in range(91, 94):
        step(t, t % 2, t % 3, True, True, True)
    step(94, 0, 1, True, True, False)
    pltpu.async_copy(src_hbm.at[pl.ds(base + NF * CH, CT)], st, pst)
    pltpu.async_copy(dst_hbm.at[pl.ds(base + NF * CH, CT)], dt, pdt)
    step(95, 1, 2, True, False, False)
    pltpu.make_async_copy(src_hbm.at[pl.ds(0, CT)], st, pst).wait()
    pltpu.async_copy(y_hbm.at[st], ra.at[pl.ds(0, CT)], ga)
    pltpu.make_async_copy(src_hbm.at[pl.ds(0, CT)], dt, pdt).wait()
    pltpu.make_async_copy(y_hbm.at[st], ra.at[pl.ds(0, CT)], ga).wait()
    pltpu.async_copy(ra.at[pl.ds(0, CT)], acc_sh.at[dt], wa, add=True)
    wait_scat(rows[1], ssem[1])
    pltpu.make_async_copy(ra.at[pl.ds(0, CT)], acc_sh.at[dt], wa).wait()
    plsc.subcore_barrier()
    pltpu.sync_copy(acc_sh.at[pl.ds(sid * RPW, RPW)], out_hbm.at[cid, sid])


_ROWS = 2048
_GRID = NP // _ROWS


def _dinv_block(deg_ref):
    d = deg_ref[0] + deg_ref[1]
    return jnp.broadcast_to(lax.rsqrt(1.0 + d), (_ROWS, D))


def _mm1_body(h_ref, w_ref, deg_ref, y1_ref):
    xw = jnp.dot(h_ref[...], w_ref[...], preferred_element_type=jnp.float32)
    y1_ref[...] = xw * _dinv_block(deg_ref)


def _mid_body(acc_ref, y_ref, deg_ref, b_ref, w_ref, y2_ref):
    dinv = _dinv_block(deg_ref)
    s = acc_ref[0] + acc_ref[1] + y_ref[...]
    x2 = jnp.sin(s * dinv + b_ref[...])
    xw = jnp.dot(x2, w_ref[...], preferred_element_type=jnp.float32)
    y2_ref[...] = xw * dinv


def _final_body(acc_ref, y_ref, deg_ref, b_ref, out_ref):
    s = acc_ref[0] + acc_ref[1] + y_ref[...]
    out_ref[...] = jnp.sin(s * _dinv_block(deg_ref) + b_ref[...])


def _row_spec():
    return pl.BlockSpec((_ROWS, D), lambda i: (i, 0))


def _acc_spec():
    return pl.BlockSpec((NC, _ROWS, D), lambda i: (0, i, 0))


def _deg_spec():
    return pl.BlockSpec((NC, _ROWS, 1), lambda i: (0, i, 0))


def _full_spec(shape):
    nd = len(shape)
    return pl.BlockSpec(shape, lambda i: (0,) * nd)


def _mm1(h2, W1, deg3):
    return pl.pallas_call(
        _mm1_body,
        grid=(_GRID,),
        in_specs=[_row_spec(), _full_spec((D, D)), _deg_spec()],
        out_specs=_row_spec(),
        out_shape=jax.ShapeDtypeStruct((NP, D), jnp.float32),
    )(h2, W1, deg3)


def _mid(acc, y1, deg3, b1, W2):
    return pl.pallas_call(
        _mid_body,
        grid=(_GRID,),
        in_specs=[_acc_spec(), _row_spec(), _deg_spec(),
                  _full_spec((1, D)), _full_spec((D, D))],
        out_specs=_row_spec(),
        out_shape=jax.ShapeDtypeStruct((NP, D), jnp.float32),
    )(acc, y1, deg3, b1, W2)


def _final(acc, y2, deg3, b2):
    return pl.pallas_call(
        _final_body,
        grid=(_GRID,),
        in_specs=[_acc_spec(), _row_spec(), _deg_spec(), _full_spec((1, D))],
        out_specs=_row_spec(),
        out_shape=jax.ShapeDtypeStruct((NP, D), jnp.float32),
    )(acc, y2, deg3, b2)


def kernel(t, h, edge_index, W1, b1, W2, b2):
    src1 = edge_index[0]
    dst1 = edge_index[1]
    dst3 = dst1.reshape(NW, NCHD, CHD)
    zeros_d = jnp.zeros((RPW, D), jnp.float32)
    zeros1 = jnp.zeros((RPW,), jnp.float32)
    ones1 = jnp.ones((CHD,), jnp.float32)
    h2 = h.reshape(N, D)
    b1r = b1.reshape(1, D)
    b2r = b2.reshape(1, D)

    deg = _deg_kernel(dst3, ones1, zeros1)
    deg3 = deg.reshape(NC, NP, 1)

    y1 = _mm1(h2, W1, deg3)
    acc1 = _spmm_kernel(src1, dst1, y1, zeros_d).reshape(NC, NP, D)
    y2 = _mid(acc1, y1, deg3, b1r, W2)
    acc2 = _spmm_kernel(src1, dst1, y2, zeros_d).reshape(NC, NP, D)
    out = _final(acc2, y2, deg3, b2r)
    return out[:N].reshape(1, N, D)
